# baseline (speedup 1.0000x reference)
import functools

import jax
import jax.numpy as jnp
from jax import lax
from jax.experimental import pallas as pl
from jax.experimental.pallas import tpu as pltpu
from jax.experimental.pallas import tpu_sc as plsc

FEAT_SIZE = 38.0
THRESHOLD = 0.5

_B = 16
_HW = 1444
_A = 5
_NC = 80

_NCORES = 2
_NSUB = 16
_NW = _NCORES * _NSUB
_TCOLS = 11
_TAIL0 = _TCOLS * 128
_TAIL = _HW - _TAIL0
_UNITS = _B * _A * _TCOLS
_UPW = -(-_UNITS // _NW)


def _sc_body(cs_hbm, conf_hbm, probs_hbm, idx_hbm, cs_v, conf_v, p_v, i_v):
    cid = lax.axis_index("c")
    sid = lax.axis_index("s")
    wid = sid * _NCORES + cid

    def unit_body(u, carry):
        gu = wid * _UPW + u

        @pl.when(gu < _UNITS)
        def _():
            ba = gu // _TCOLS
            tc = gu % _TCOLS
            b = ba // _A
            a = ba % _A
            h0 = tc * 128
            pltpu.sync_copy(cs_hbm.at[b, a, :, pl.ds(h0, 128)], cs_v)
            pltpu.sync_copy(conf_hbm.at[b, a, pl.ds(h0, 128)], conf_v)

            def lane_group(l, carry2):
                cv = conf_v[pl.ds(l * 16, 16)]

                def cbody(c, mc):
                    m, am = mc
                    s = cs_v[c, pl.ds(l * 16, 16)] * cv
                    p = s > m
                    return (jnp.where(p, s, m),
                            jnp.where(p, jnp.zeros((16,), jnp.int32) + c, am))

                m0 = jnp.full((16,), -1.0, jnp.float32)
                am0 = jnp.zeros((16,), jnp.int32)
                m, am = lax.fori_loop(0, _NC, cbody, (m0, am0), unroll=8)
                pm = m > THRESHOLD
                p_v[pl.ds(l * 16, 16)] = jnp.where(pm, m, 0.0)
                i_v[pl.ds(l * 16, 16)] = am
                return carry2

            lax.fori_loop(0, 8, lane_group, 0)
            pltpu.sync_copy(p_v, probs_hbm.at[b, a, pl.ds(h0, 128)])
            pltpu.sync_copy(i_v, idx_hbm.at[b, a, pl.ds(h0, 128)])

        return carry

    lax.fori_loop(0, _UPW, unit_body, 0)


def _sc_call(cs_t, conf_t):
    mesh = plsc.VectorSubcoreMesh(core_axis_name="c", subcore_axis_name="s")
    fn = functools.partial(
        pl.kernel,
        out_type=[
            jax.ShapeDtypeStruct((_B, _A, _HW), jnp.float32),
            jax.ShapeDtypeStruct((_B, _A, _HW), jnp.int32),
        ],
        mesh=mesh,
        scratch_types=[
            pltpu.VMEM((_NC, 128), jnp.float32),
            pltpu.VMEM((128,), jnp.float32),
            pltpu.VMEM((128,), jnp.float32),
            pltpu.VMEM((128,), jnp.int32),
        ],
        compiler_params=pltpu.CompilerParams(use_tc_tiling_on_sc=True),
    )(_sc_body)
    return fn(cs_t, conf_t)


def _tc_body(probs_ref, idx_ref, cst_ref, cft_ref, box_ref, prior_ref,
             boxo_ref, probso_ref, idxo_ref):
    cst = cst_ref[...]
    cft = cft_ref[...]
    st = cst * cft[:, :, None, :]
    mt = jnp.max(st, axis=2)
    iota = lax.broadcasted_iota(jnp.int32, st.shape, 2).astype(jnp.float32)
    amt = jnp.min(jnp.where(st == mt[:, :, None, :], iota, 128.0),
                  axis=2).astype(jnp.int32)
    pt = jnp.where(mt > THRESHOLD, mt, 0.0)

    p_in = probs_ref[...]
    i_in = idx_ref[...]
    probs = jnp.concatenate([p_in[:, :, :_TAIL0], pt[:, :, :_TAIL]], axis=2)
    idx = jnp.concatenate([i_in[:, :, :_TAIL0], amt[:, :, :_TAIL]], axis=2)
    mask = probs > THRESHOLD

    box = box_ref[...]
    prior = prior_ref[...]
    xy = box[:, :, :2, :] + prior[None, :, :2, :]
    wh = box[:, :, 2:, :] * prior[None, :, 2:, :]
    mins = xy - wh / 2.0
    maxs = xy + wh / 2.0
    corners = jnp.concatenate([mins, maxs], axis=2) / FEAT_SIZE
    boxo_ref[...] = jnp.where(mask[:, :, None, :], corners, 0.0)
    probso_ref[...] = probs
    idxo_ref[...] = idx


def kernel(box, box_confidence, class_score, prior):
    cs_t = jnp.transpose(class_score, (0, 2, 3, 1))
    conf_t = jnp.transpose(box_confidence[..., 0], (0, 2, 1))
    box_t = jnp.transpose(box, (0, 2, 3, 1))
    prior_t = jnp.transpose(prior, (1, 2, 0))

    probs_sc, idx_sc = _sc_call(cs_t, conf_t)

    boxo_t, probs_t, idx_t = pl.pallas_call(
        _tc_body,
        grid=(_B,),
        in_specs=[
            pl.BlockSpec((1, _A, _HW), lambda b: (b, 0, 0)),
            pl.BlockSpec((1, _A, _HW), lambda b: (b, 0, 0)),
            pl.BlockSpec((1, _A, _NC, 128), lambda b: (b, 0, 0, _TCOLS)),
            pl.BlockSpec((1, _A, 128), lambda b: (b, 0, _TCOLS)),
            pl.BlockSpec((1, _A, 4, _HW), lambda b: (b, 0, 0, 0)),
            pl.BlockSpec((_A, 4, _HW), lambda b: (0, 0, 0)),
        ],
        out_specs=[
            pl.BlockSpec((1, _A, 4, _HW), lambda b: (b, 0, 0, 0)),
            pl.BlockSpec((1, _A, _HW), lambda b: (b, 0, 0)),
            pl.BlockSpec((1, _A, _HW), lambda b: (b, 0, 0)),
        ],
        out_shape=[
            jax.ShapeDtypeStruct((_B, _A, 4, _HW), jnp.float32),
            jax.ShapeDtypeStruct((_B, _A, _HW), jnp.float32),
            jax.ShapeDtypeStruct((_B, _A, _HW), jnp.int32),
        ],
        compiler_params=pltpu.CompilerParams(
            dimension_semantics=("parallel",),
        ),
    )(probs_sc, idx_sc, cs_t, conf_t, box_t, prior_t)

    box_out = jnp.transpose(boxo_t, (0, 3, 1, 2))
    probs_out = jnp.transpose(probs_t, (0, 2, 1))
    idx_out = jnp.transpose(idx_t, (0, 2, 1))
    return box_out, probs_out, idx_out


# speedup vs baseline: 2.0985x; 2.0985x over previous
import functools

import jax
import jax.numpy as jnp
from jax import lax
from jax.experimental import pallas as pl
from jax.experimental.pallas import tpu as pltpu
from jax.experimental.pallas import tpu_sc as plsc

FEAT_SIZE = 38.0
THRESHOLD = 0.5

_B = 16
_SPLIT = 12
_BHI = _B - _SPLIT
_HW = 1444
_A = 5
_NC = 80

_NCORES = 2
_NSUB = 16
_NW = _NCORES * _NSUB
_TCOLS = 11
_TAIL0 = _TCOLS * 128
_TAIL = _HW - _TAIL0
_UNITS = _BHI * _A * _TCOLS
_UPW = -(-_UNITS // _NW)


def _sc_body(cs_hbm, conf_hbm, probs_hbm, idx_hbm, cs_v, conf_v, p_v, i_v):
    cid = lax.axis_index("c")
    sid = lax.axis_index("s")
    wid = sid * _NCORES + cid

    def unit_body(u, carry):
        gu = wid * _UPW + u

        @pl.when(gu < _UNITS)
        def _():
            ba = gu // _TCOLS
            tc = gu % _TCOLS
            bl = ba // _A
            a = ba % _A
            h0 = tc * 128
            pltpu.sync_copy(cs_hbm.at[_SPLIT + bl, a, :, pl.ds(h0, 128)], cs_v)
            pltpu.sync_copy(conf_hbm.at[_SPLIT + bl, a, pl.ds(h0, 128)], conf_v)

            def lane_group(l, carry2):
                cv = conf_v[pl.ds(l * 16, 16)]

                def cbody(c, mc):
                    m, am = mc
                    s = cs_v[c, pl.ds(l * 16, 16)] * cv
                    p = s > m
                    return (jnp.where(p, s, m),
                            jnp.where(p, jnp.zeros((16,), jnp.int32) + c, am))

                m0 = jnp.full((16,), -1.0, jnp.float32)
                am0 = jnp.zeros((16,), jnp.int32)
                m, am = lax.fori_loop(0, _NC, cbody, (m0, am0), unroll=8)
                pm = m > THRESHOLD
                p_v[pl.ds(l * 16, 16)] = jnp.where(pm, m, 0.0)
                i_v[pl.ds(l * 16, 16)] = am
                return carry2

            lax.fori_loop(0, 8, lane_group, 0)
            pltpu.sync_copy(p_v, probs_hbm.at[bl, a, pl.ds(h0, 128)])
            pltpu.sync_copy(i_v, idx_hbm.at[bl, a, pl.ds(h0, 128)])

        return carry

    lax.fori_loop(0, _UPW, unit_body, 0)


def _sc_call(cs_t, conf_t):
    mesh = plsc.VectorSubcoreMesh(core_axis_name="c", subcore_axis_name="s")
    fn = functools.partial(
        pl.kernel,
        out_type=[
            jax.ShapeDtypeStruct((_BHI, _A, _HW), jnp.float32),
            jax.ShapeDtypeStruct((_BHI, _A, _HW), jnp.int32),
        ],
        mesh=mesh,
        scratch_types=[
            pltpu.VMEM((_NC, 128), jnp.float32),
            pltpu.VMEM((128,), jnp.float32),
            pltpu.VMEM((128,), jnp.float32),
            pltpu.VMEM((128,), jnp.int32),
        ],
        compiler_params=pltpu.CompilerParams(use_tc_tiling_on_sc=True),
    )(_sc_body)
    return fn(cs_t, conf_t)


def _decode_box(box, prior, mask):
    xy = box[:, :, :2, :] + prior[None, :, :2, :]
    wh = box[:, :, 2:, :] * prior[None, :, 2:, :]
    mins = xy - wh / 2.0
    maxs = xy + wh / 2.0
    corners = jnp.concatenate([mins, maxs], axis=2) / FEAT_SIZE
    return jnp.where(mask[:, :, None, :], corners, 0.0)


def _tc_lo_body(cs_ref, conf_ref, box_ref, prior_ref,
                boxo_ref, probs_ref, idx_ref):
    cs = cs_ref[...]
    conf = conf_ref[...]
    scores = cs * conf[:, :, None, :]
    m = jnp.max(scores, axis=2)
    iota = lax.broadcasted_iota(jnp.int32, scores.shape, 2).astype(jnp.float32)
    amf = jnp.min(jnp.where(scores == m[:, :, None, :], iota, 128.0), axis=2)
    am = amf.astype(jnp.int32)
    mask = m > THRESHOLD
    boxo_ref[...] = _decode_box(box_ref[...], prior_ref[...], mask)
    probs_ref[...] = jnp.where(mask, m, 0.0)
    idx_ref[...] = am


def _tc_hi_body(probs_ref, idx_ref, cst_ref, cft_ref, box_ref, prior_ref,
                boxo_ref, probso_ref, idxo_ref):
    cst = cst_ref[...]
    cft = cft_ref[...]
    st = cst * cft[:, :, None, :]
    mt = jnp.max(st, axis=2)
    iota = lax.broadcasted_iota(jnp.int32, st.shape, 2).astype(jnp.float32)
    amt = jnp.min(jnp.where(st == mt[:, :, None, :], iota, 128.0),
                  axis=2).astype(jnp.int32)
    pt = jnp.where(mt > THRESHOLD, mt, 0.0)

    p_in = probs_ref[...]
    i_in = idx_ref[...]
    probs = jnp.concatenate([p_in[:, :, :_TAIL0], pt[:, :, :_TAIL]], axis=2)
    idx = jnp.concatenate([i_in[:, :, :_TAIL0], amt[:, :, :_TAIL]], axis=2)
    mask = probs > THRESHOLD
    boxo_ref[...] = _decode_box(box_ref[...], prior_ref[...], mask)
    probso_ref[...] = probs
    idxo_ref[...] = idx


def kernel(box, box_confidence, class_score, prior):
    cs_t = jnp.transpose(class_score, (0, 2, 3, 1))
    conf_t = jnp.transpose(box_confidence[..., 0], (0, 2, 1))
    box_t = jnp.transpose(box, (0, 2, 3, 1))
    prior_t = jnp.transpose(prior, (1, 2, 0))

    probs_sc, idx_sc = _sc_call(cs_t, conf_t)

    boxo_lo, probs_lo, idx_lo = pl.pallas_call(
        _tc_lo_body,
        grid=(_SPLIT,),
        in_specs=[
            pl.BlockSpec((1, _A, _NC, _HW), lambda b: (b, 0, 0, 0)),
            pl.BlockSpec((1, _A, _HW), lambda b: (b, 0, 0)),
            pl.BlockSpec((1, _A, 4, _HW), lambda b: (b, 0, 0, 0)),
            pl.BlockSpec((_A, 4, _HW), lambda b: (0, 0, 0)),
        ],
        out_specs=[
            pl.BlockSpec((1, _A, 4, _HW), lambda b: (b, 0, 0, 0)),
            pl.BlockSpec((1, _A, _HW), lambda b: (b, 0, 0)),
            pl.BlockSpec((1, _A, _HW), lambda b: (b, 0, 0)),
        ],
        out_shape=[
            jax.ShapeDtypeStruct((_SPLIT, _A, 4, _HW), jnp.float32),
            jax.ShapeDtypeStruct((_SPLIT, _A, _HW), jnp.float32),
            jax.ShapeDtypeStruct((_SPLIT, _A, _HW), jnp.int32),
        ],
        compiler_params=pltpu.CompilerParams(
            dimension_semantics=("parallel",),
        ),
    )(cs_t, conf_t, box_t, prior_t)

    boxo_hi, probs_hi, idx_hi = pl.pallas_call(
        _tc_hi_body,
        grid=(_BHI,),
        in_specs=[
            pl.BlockSpec((1, _A, _HW), lambda b: (b, 0, 0)),
            pl.BlockSpec((1, _A, _HW), lambda b: (b, 0, 0)),
            pl.BlockSpec((1, _A, _NC, 128), lambda b: (b + _SPLIT, 0, 0, _TCOLS)),
            pl.BlockSpec((1, _A, 128), lambda b: (b + _SPLIT, 0, _TCOLS)),
            pl.BlockSpec((1, _A, 4, _HW), lambda b: (b + _SPLIT, 0, 0, 0)),
            pl.BlockSpec((_A, 4, _HW), lambda b: (0, 0, 0)),
        ],
        out_specs=[
            pl.BlockSpec((1, _A, 4, _HW), lambda b: (b, 0, 0, 0)),
            pl.BlockSpec((1, _A, _HW), lambda b: (b, 0, 0)),
            pl.BlockSpec((1, _A, _HW), lambda b: (b, 0, 0)),
        ],
        out_shape=[
            jax.ShapeDtypeStruct((_BHI, _A, 4, _HW), jnp.float32),
            jax.ShapeDtypeStruct((_BHI, _A, _HW), jnp.float32),
            jax.ShapeDtypeStruct((_BHI, _A, _HW), jnp.int32),
        ],
        compiler_params=pltpu.CompilerParams(
            dimension_semantics=("parallel",),
        ),
    )(probs_sc, idx_sc, cs_t, conf_t, box_t, prior_t)

    boxo_t = jnp.concatenate([boxo_lo, boxo_hi], axis=0)
    probs_t = jnp.concatenate([probs_lo, probs_hi], axis=0)
    idx_t = jnp.concatenate([idx_lo, idx_hi], axis=0)

    box_out = jnp.transpose(boxo_t, (0, 3, 1, 2))
    probs_out = jnp.transpose(probs_t, (0, 2, 1))
    idx_out = jnp.transpose(idx_t, (0, 2, 1))
    return box_out, probs_out, idx_out


# speedup vs baseline: 3.5730x; 1.7026x over previous
import jax
import jax.numpy as jnp
from jax.experimental import pallas as pl
from jax.experimental.pallas import tpu as pltpu

FEAT_SIZE = 38.0
THRESHOLD = 0.5

_B = 16
_HW = 1444
_A = 5
_NC = 80
_HB = 1444
_NBH = _HW // _HB


def _body(cs_ref, conf_ref, box_ref, prior_ref, boxo_ref, probs_ref, idx_ref):
    cs = cs_ref[...]
    conf = conf_ref[...]
    scores = cs * conf[:, :, None, :]
    m = jnp.max(scores, axis=2)
    iota = jax.lax.broadcasted_iota(jnp.int32, scores.shape, 2).astype(jnp.float32)
    amf = jnp.min(jnp.where(scores == m[:, :, None, :], iota, 128.0), axis=2)
    am = amf.astype(jnp.int32)
    mask = m > THRESHOLD

    box = box_ref[...]
    prior = prior_ref[...]
    xy = box[:, :, :2, :] + prior[None, :, :2, :]
    wh = box[:, :, 2:, :] * prior[None, :, 2:, :]
    mins = xy - wh / 2.0
    maxs = xy + wh / 2.0
    corners = jnp.concatenate([mins, maxs], axis=2) / FEAT_SIZE
    boxo_ref[...] = jnp.where(mask[:, :, None, :], corners, 0.0)
    probs_ref[...] = jnp.where(mask, m, 0.0)
    idx_ref[...] = am


def kernel(box, box_confidence, class_score, prior):
    cs_t = jnp.transpose(class_score, (0, 2, 3, 1))
    conf_t = jnp.transpose(box_confidence[..., 0], (0, 2, 1))
    box_t = jnp.transpose(box, (0, 2, 3, 1))
    prior_t = jnp.transpose(prior, (1, 2, 0))

    boxo_t, probs_t, idx_t = pl.pallas_call(
        _body,
        grid=(_B, _NBH),
        in_specs=[
            pl.BlockSpec((1, _A, _NC, _HB), lambda b, h: (b, 0, 0, h)),
            pl.BlockSpec((1, _A, _HB), lambda b, h: (b, 0, h)),
            pl.BlockSpec((1, _A, 4, _HB), lambda b, h: (b, 0, 0, h)),
            pl.BlockSpec((_A, 4, _HB), lambda b, h: (0, 0, h)),
        ],
        out_specs=[
            pl.BlockSpec((1, _A, 4, _HB), lambda b, h: (b, 0, 0, h)),
            pl.BlockSpec((1, _A, _HB), lambda b, h: (b, 0, h)),
            pl.BlockSpec((1, _A, _HB), lambda b, h: (b, 0, h)),
        ],
        out_shape=[
            jax.ShapeDtypeStruct((_B, _A, 4, _HW), jnp.float32),
            jax.ShapeDtypeStruct((_B, _A, _HW), jnp.float32),
            jax.ShapeDtypeStruct((_B, _A, _HW), jnp.int32),
        ],
        compiler_params=pltpu.CompilerParams(
            dimension_semantics=("parallel", "parallel"),
        ),
    )(cs_t, conf_t, box_t, prior_t)

    box_out = jnp.transpose(boxo_t, (0, 3, 1, 2))
    probs_out = jnp.transpose(probs_t, (0, 2, 1))
    idx_out = jnp.transpose(idx_t, (0, 2, 1))
    return box_out, probs_out, idx_out
